# baseline (speedup 1.0000x reference)
import functools

import jax
import jax.numpy as jnp
from jax import lax
from jax.experimental import pallas as pl
from jax.experimental.pallas import tpu as pltpu
from jax.experimental.pallas import tpu_sc as plsc


def kernel(logits, noise, x, mask, unmask_idx):
    B, F, V = logits.shape
    L = unmask_idx.shape[1]

    logits_t = jnp.transpose(logits, (1, 2, 0))
    noise_t = jnp.transpose(noise, (1, 2, 0))
    x_t = x.T.reshape(F, 1, B)
    mask_t = mask.T.astype(jnp.int32).reshape(F, 1, B)
    umi_t = unmask_idx.T.reshape(L, 1, B)

    def tc_body(lg_ref, ns_ref, xt_ref, mt_ref, umi_ref, newx_ref):
        f = pl.program_id(0)
        val = lg_ref[0] + ns_ref[0]
        maxv = jnp.max(val, axis=0)
        iota_v = lax.broadcasted_iota(jnp.int32, (V, B), 0)
        amax = jnp.min(jnp.where(val == maxv[None, :], iota_v, V), axis=0)
        m = mt_ref[0, 0]
        hit = (umi_ref[0, 0] == f)
        for l in range(1, L):
            hit = hit | (umi_ref[l, 0] == f)
        diff = hit & (m == 0)
        newx_ref[0, 0] = jnp.where(diff, amax, xt_ref[0, 0])

    new_x_t = pl.pallas_call(
        tc_body,
        grid=(F,),
        in_specs=[
            pl.BlockSpec((1, V, B), lambda f: (f, 0, 0)),
            pl.BlockSpec((1, V, B), lambda f: (f, 0, 0)),
            pl.BlockSpec((1, 1, B), lambda f: (f, 0, 0)),
            pl.BlockSpec((1, 1, B), lambda f: (f, 0, 0)),
            pl.BlockSpec((L, 1, B), lambda f: (0, 0, 0)),
        ],
        out_specs=pl.BlockSpec((1, 1, B), lambda f: (f, 0, 0)),
        out_shape=jax.ShapeDtypeStruct((F, 1, B), jnp.int32),
        compiler_params=pltpu.CompilerParams(
            dimension_semantics=("arbitrary",)),
    )(logits_t, noise_t, x_t, mask_t, umi_t)
    new_x = new_x_t.reshape(F, B).T

    info = plsc.get_sparse_core_info()
    NC, NS, LN = info.num_cores, info.num_subcores, info.num_lanes
    NW = NC * NS
    assert B % NW == 0
    PW = B // NW
    SPAN = PW * F
    NIDX = PW * L
    assert NIDX % LN == 0 and SPAN % LN == 0 and SPAN % 8 == 0 and NIDX % 8 == 0
    assert L & (L - 1) == 0
    LSH = L.bit_length() - 1
    NCHUNK = NIDX // LN

    mask_flat = mask.astype(jnp.int32).reshape(-1)
    umi_flat = unmask_idx.reshape(-1)

    @functools.partial(
        pl.kernel,
        mesh=plsc.VectorSubcoreMesh(core_axis_name="c", subcore_axis_name="s"),
        compiler_params=pltpu.CompilerParams(
            needs_layout_passes=False, use_tc_tiling_on_sc=False),
        out_type=(
            jax.ShapeDtypeStruct((B * F,), jnp.int32),
            jax.ShapeDtypeStruct((B * F,), jnp.float32),
        ),
        scratch_types=[
            pltpu.VMEM((SPAN,), jnp.int32),
            pltpu.VMEM((SPAN,), jnp.int32),
            pltpu.VMEM((SPAN,), jnp.float32),
            pltpu.VMEM((NIDX,), jnp.int32),
        ],
    )
    def update_kernel(mask_hbm, umi_hbm, newmask_hbm, fmask_hbm,
                      mv, nmv, fmv, umi_v):
        wid = lax.axis_index("s") * NC + lax.axis_index("c")
        base = wid * SPAN
        ibase = wid * NIDX
        pltpu.sync_copy(mask_hbm.at[pl.ds(base, SPAN)], mv)
        pltpu.sync_copy(umi_hbm.at[pl.ds(ibase, NIDX)], umi_v)

        lanes = lax.iota(jnp.int32, LN)

        for i in range(SPAN // LN):
            sl = pl.ds(i * LN, LN)
            m = mv[sl]
            fmv[sl] = jnp.where(m != 0, 0.0, -jnp.inf).astype(jnp.float32)
            nmv[sl] = m

        ones = jnp.ones((LN,), jnp.int32)
        for j in range(NCHUNK):
            uv = umi_v[pl.ds(j * LN, LN)]
            bloc = lax.shift_right_logical(j * LN + lanes, LSH)
            locidx = bloc * F + uv
            plsc.store_scatter(nmv, [locidx], ones)

        pltpu.sync_copy(nmv, newmask_hbm.at[pl.ds(base, SPAN)])
        pltpu.sync_copy(fmv, fmask_hbm.at[pl.ds(base, SPAN)])

    new_mask_flat, fmask_flat = update_kernel(mask_flat, umi_flat)
    return (new_x,
            new_mask_flat.reshape(B, F).astype(bool),
            fmask_flat.reshape(B, F))


# speedup vs baseline: 1.0514x; 1.0514x over previous
import functools

import jax
import jax.numpy as jnp
from jax import lax
from jax.experimental import pallas as pl
from jax.experimental.pallas import tpu as pltpu
from jax.experimental.pallas import tpu_sc as plsc


def kernel(logits, noise, x, mask, unmask_idx):
    B, F, V = logits.shape
    L = unmask_idx.shape[1]

    logits_t = jnp.transpose(logits, (1, 2, 0))
    noise_t = jnp.transpose(noise, (1, 2, 0))
    x_t = x.T
    mask_t = mask.T.astype(jnp.int32)
    umi_t = unmask_idx.T

    def tc_body(lg_ref, ns_ref, xt_ref, mt_ref, umi_ref, newx_ref):
        f = pl.program_id(0)
        val = lg_ref[0] + ns_ref[0]
        maxv = jnp.max(val, axis=0)
        iota_v = lax.broadcasted_iota(jnp.int32, (V, B), 0)
        amax = jnp.min(jnp.where(val == maxv[None, :], iota_v, V), axis=0)
        m = mt_ref[pl.ds(f, 1), :]
        hit = (umi_ref[pl.ds(0, 1), :] == f)
        for l in range(1, L):
            hit = hit | (umi_ref[pl.ds(l, 1), :] == f)
        diff = hit & (m == 0)
        newx_ref[pl.ds(f, 1), :] = jnp.where(
            diff, amax[None, :], xt_ref[pl.ds(f, 1), :])

    new_x_t = pl.pallas_call(
        tc_body,
        grid=(F,),
        in_specs=[
            pl.BlockSpec((1, V, B), lambda f: (f, 0, 0)),
            pl.BlockSpec((1, V, B), lambda f: (f, 0, 0)),
            pl.BlockSpec((F, B), lambda f: (0, 0)),
            pl.BlockSpec((F, B), lambda f: (0, 0)),
            pl.BlockSpec((L, B), lambda f: (0, 0)),
        ],
        out_specs=pl.BlockSpec((F, B), lambda f: (0, 0)),
        out_shape=jax.ShapeDtypeStruct((F, B), jnp.int32),
        compiler_params=pltpu.CompilerParams(
            dimension_semantics=("arbitrary",)),
    )(logits_t, noise_t, x_t, mask_t, umi_t)
    new_x = new_x_t.T

    info = plsc.get_sparse_core_info()
    NC, NS, LN = info.num_cores, info.num_subcores, info.num_lanes
    NW = NC * NS
    assert B % NW == 0
    PW = B // NW
    SPAN = PW * F
    NIDX = PW * L
    assert NIDX % LN == 0 and SPAN % LN == 0 and SPAN % 8 == 0 and NIDX % 8 == 0
    assert L & (L - 1) == 0
    LSH = L.bit_length() - 1
    NCHUNK = NIDX // LN

    mask_flat = mask.astype(jnp.int32).reshape(-1)
    umi_flat = unmask_idx.reshape(-1)

    @functools.partial(
        pl.kernel,
        mesh=plsc.VectorSubcoreMesh(core_axis_name="c", subcore_axis_name="s"),
        compiler_params=pltpu.CompilerParams(
            needs_layout_passes=False, use_tc_tiling_on_sc=False),
        out_type=(
            jax.ShapeDtypeStruct((B * F,), jnp.int32),
            jax.ShapeDtypeStruct((B * F,), jnp.float32),
        ),
        scratch_types=[
            pltpu.VMEM((SPAN,), jnp.int32),
            pltpu.VMEM((SPAN,), jnp.int32),
            pltpu.VMEM((SPAN,), jnp.float32),
            pltpu.VMEM((NIDX,), jnp.int32),
        ],
    )
    def update_kernel(mask_hbm, umi_hbm, newmask_hbm, fmask_hbm,
                      mv, nmv, fmv, umi_v):
        wid = lax.axis_index("s") * NC + lax.axis_index("c")
        base = wid * SPAN
        ibase = wid * NIDX
        pltpu.sync_copy(mask_hbm.at[pl.ds(base, SPAN)], mv)
        pltpu.sync_copy(umi_hbm.at[pl.ds(ibase, NIDX)], umi_v)

        lanes = lax.iota(jnp.int32, LN)

        for i in range(SPAN // LN):
            sl = pl.ds(i * LN, LN)
            m = mv[sl]
            fmv[sl] = jnp.where(m != 0, 0.0, -jnp.inf).astype(jnp.float32)
            nmv[sl] = m

        ones = jnp.ones((LN,), jnp.int32)
        for j in range(NCHUNK):
            uv = umi_v[pl.ds(j * LN, LN)]
            bloc = lax.shift_right_logical(j * LN + lanes, LSH)
            locidx = bloc * F + uv
            plsc.store_scatter(nmv, [locidx], ones)

        pltpu.sync_copy(nmv, newmask_hbm.at[pl.ds(base, SPAN)])
        pltpu.sync_copy(fmv, fmask_hbm.at[pl.ds(base, SPAN)])

    new_mask_flat, fmask_flat = update_kernel(mask_flat, umi_flat)
    return (new_x,
            new_mask_flat.reshape(B, F).astype(bool),
            fmask_flat.reshape(B, F))


# speedup vs baseline: 1.3679x; 1.3010x over previous
import functools

import jax
import jax.numpy as jnp
from jax import lax
from jax.experimental import pallas as pl
from jax.experimental.pallas import tpu as pltpu
from jax.experimental.pallas import tpu_sc as plsc


def kernel(logits, noise, x, mask, unmask_idx):
    B, F, V = logits.shape
    L = unmask_idx.shape[1]

    logits_t = jnp.transpose(logits, (1, 2, 0))
    noise_t = jnp.transpose(noise, (1, 2, 0))
    x_t = x.T
    mask_t = mask.T.astype(jnp.int32)
    umi_t = unmask_idx.T

    def tc_body(lg_ref, ns_ref, xt_ref, mt_ref, umi_ref,
                newx_ref, newm_ref, fm_ref):
        f = pl.program_id(0)
        val = lg_ref[0] + ns_ref[0]
        maxv = jnp.max(val, axis=0)
        iota_v = lax.broadcasted_iota(jnp.int32, (V, B), 0)
        amax = jnp.min(jnp.where(val == maxv[None, :], iota_v, V), axis=0)
        m = mt_ref[pl.ds(f, 1), :]
        hit = (umi_ref[pl.ds(0, 1), :] == f)
        for l in range(1, L):
            hit = hit | (umi_ref[pl.ds(l, 1), :] == f)
        diff = hit & (m == 0)
        newx_ref[pl.ds(f, 1), :] = jnp.where(
            diff, amax[None, :], xt_ref[pl.ds(f, 1), :])
        newm_ref[pl.ds(f, 1), :] = jnp.where(hit, 1, m)
        fm_ref[pl.ds(f, 1), :] = jnp.where(m != 0, 0.0, -jnp.inf)

    new_x_t, new_mask_t, fm_t = pl.pallas_call(
        tc_body,
        grid=(F,),
        in_specs=[
            pl.BlockSpec((1, V, B), lambda f: (f, 0, 0)),
            pl.BlockSpec((1, V, B), lambda f: (f, 0, 0)),
            pl.BlockSpec((F, B), lambda f: (0, 0)),
            pl.BlockSpec((F, B), lambda f: (0, 0)),
            pl.BlockSpec((L, B), lambda f: (0, 0)),
        ],
        out_specs=[
            pl.BlockSpec((F, B), lambda f: (0, 0)),
            pl.BlockSpec((F, B), lambda f: (0, 0)),
            pl.BlockSpec((F, B), lambda f: (0, 0)),
        ],
        out_shape=[
            jax.ShapeDtypeStruct((F, B), jnp.int32),
            jax.ShapeDtypeStruct((F, B), jnp.int32),
            jax.ShapeDtypeStruct((F, B), jnp.float32),
        ],
        compiler_params=pltpu.CompilerParams(
            dimension_semantics=("arbitrary",)),
    )(logits_t, noise_t, x_t, mask_t, umi_t)
    new_x = new_x_t.T
    return new_x, new_mask_t.T.astype(bool), fm_t.T

    info = plsc.get_sparse_core_info()
    NC, NS, LN = info.num_cores, info.num_subcores, info.num_lanes
    NW = NC * NS
    assert B % NW == 0
    PW = B // NW
    SPAN = PW * F
    NIDX = PW * L
    assert NIDX % LN == 0 and SPAN % LN == 0 and SPAN % 8 == 0 and NIDX % 8 == 0
    assert L & (L - 1) == 0
    LSH = L.bit_length() - 1
    NCHUNK = NIDX // LN

    mask_flat = mask.astype(jnp.int32).reshape(-1)
    umi_flat = unmask_idx.reshape(-1)

    @functools.partial(
        pl.kernel,
        mesh=plsc.VectorSubcoreMesh(core_axis_name="c", subcore_axis_name="s"),
        compiler_params=pltpu.CompilerParams(
            needs_layout_passes=False, use_tc_tiling_on_sc=False),
        out_type=(
            jax.ShapeDtypeStruct((B * F,), jnp.int32),
            jax.ShapeDtypeStruct((B * F,), jnp.float32),
        ),
        scratch_types=[
            pltpu.VMEM((SPAN,), jnp.int32),
            pltpu.VMEM((SPAN,), jnp.int32),
            pltpu.VMEM((SPAN,), jnp.float32),
            pltpu.VMEM((NIDX,), jnp.int32),
        ],
    )
    def update_kernel(mask_hbm, umi_hbm, newmask_hbm, fmask_hbm,
                      mv, nmv, fmv, umi_v):
        wid = lax.axis_index("s") * NC + lax.axis_index("c")
        base = wid * SPAN
        ibase = wid * NIDX
        pltpu.sync_copy(mask_hbm.at[pl.ds(base, SPAN)], mv)
        pltpu.sync_copy(umi_hbm.at[pl.ds(ibase, NIDX)], umi_v)

        lanes = lax.iota(jnp.int32, LN)

        for i in range(SPAN // LN):
            sl = pl.ds(i * LN, LN)
            m = mv[sl]
            fmv[sl] = jnp.where(m != 0, 0.0, -jnp.inf).astype(jnp.float32)
            nmv[sl] = m

        ones = jnp.ones((LN,), jnp.int32)
        for j in range(NCHUNK):
            uv = umi_v[pl.ds(j * LN, LN)]
            bloc = lax.shift_right_logical(j * LN + lanes, LSH)
            locidx = bloc * F + uv
            plsc.store_scatter(nmv, [locidx], ones)

        pltpu.sync_copy(nmv, newmask_hbm.at[pl.ds(base, SPAN)])
        pltpu.sync_copy(fmv, fmask_hbm.at[pl.ds(base, SPAN)])

    new_mask_flat, fmask_flat = update_kernel(mask_flat, umi_flat)
    return (new_x,
            new_mask_flat.reshape(B, F).astype(bool),
            fmask_flat.reshape(B, F))
